# baseline (speedup 1.0000x reference)
import functools

import jax
import jax.numpy as jnp
from jax import lax
from jax.experimental import pallas as pl
from jax.experimental.pallas import tpu as pltpu
from jax.experimental.pallas import tpu_sc as plsc

_N = 10000
_E = 320000
_D = 128
_H = 256
_L = 3

_C = 128
_NCH = _E // _C
_NSUB = 16
_NCORE = 2
_RPS = 624
_REM = _N - _RPS * _NSUB


def _graph_ln(h, w, b, eps=1e-5):
    mu = jnp.mean(h)
    var = jnp.mean((h - mu) ** 2)
    return (h - mu) / (jnp.sqrt(var) + eps) * w + b


def _emit_tables(h, lnw, lnb, t, z_ref, tab_ref):
    z = jnp.maximum(_graph_ln(h, lnw, lnb), 0.0)
    z_ref[...] = z
    msg = z + 1e-7
    a = jnp.exp(msg * t)
    tab_ref[0:_N, :] = a
    tab_ref[_N : 2 * _N, :] = msg * a


def _conv_mlp(h_ref, z_ref, s_ref, w1_ref, b1_ref, lnw_ref, lnb_ref,
              w2_ref, b2_ref):
    s1 = s_ref[0:_N, :]
    s2 = s_ref[_N : 2 * _N, :]
    out = s2 / (s1 + 1e-16) + z_ref[...]
    h1 = (
        jnp.dot(out, w1_ref[...], preferred_element_type=jnp.float32)
        + b1_ref[...]
    )
    g = jnp.maximum(_graph_ln(h1, lnw_ref[...], lnb_ref[...]), 0.0)
    return (
        h_ref[...]
        + jnp.dot(g, w2_ref[...], preferred_element_type=jnp.float32)
        + b2_ref[...]
    )


def _fin_body(x_ref, w_ref, b_ref, lnw_ref, lnb_ref, t_ref,
              h_ref, z_ref, tab_ref):
    h = (
        jnp.dot(x_ref[...], w_ref[...], preferred_element_type=jnp.float32)
        + b_ref[...]
    )
    h_ref[...] = h
    _emit_tables(h, lnw_ref[...], lnb_ref[...], t_ref[0, 0], z_ref, tab_ref)


def _fmid_body(h_ref, z_ref, s_ref, w1_ref, b1_ref, lnw_ref, lnb_ref,
               w2_ref, b2_ref, lnwn_ref, lnbn_ref, tn_ref,
               ho_ref, zo_ref, tab_ref):
    h = _conv_mlp(h_ref, z_ref, s_ref, w1_ref, b1_ref, lnw_ref, lnb_ref,
                  w2_ref, b2_ref)
    ho_ref[...] = h
    _emit_tables(h, lnwn_ref[...], lnbn_ref[...], tn_ref[0, 0], zo_ref,
                 tab_ref)


def _fout_body(h_ref, z_ref, s_ref, w1_ref, b1_ref, lnw_ref, lnb_ref,
               w2_ref, b2_ref, lw_ref, lb_ref, ow_ref, ob_ref, o_ref):
    h = _conv_mlp(h_ref, z_ref, s_ref, w1_ref, b1_ref, lnw_ref, lnb_ref,
                  w2_ref, b2_ref)
    g = jnp.maximum(
        jnp.dot(h, lw_ref[...], preferred_element_type=jnp.float32)
        + lb_ref[...],
        0.0,
    )
    o_ref[...] = (
        jnp.dot(g, ow_ref[...], preferred_element_type=jnp.float32)
        + ob_ref[...]
    )


_hzt_shapes = (
    jax.ShapeDtypeStruct((_N, _D), jnp.float32),
    jax.ShapeDtypeStruct((_N, _D), jnp.float32),
    jax.ShapeDtypeStruct((2 * _N, _D), jnp.float32),
)

_fin = pl.pallas_call(_fin_body, out_shape=_hzt_shapes)
_fmid = pl.pallas_call(_fmid_body, out_shape=_hzt_shapes)
_fout = pl.pallas_call(
    _fout_body, out_shape=jax.ShapeDtypeStruct((_N, _D), jnp.float32)
)


_sc_mesh = plsc.VectorSubcoreMesh(
    core_axis_name="c", subcore_axis_name="s", num_cores=_NCORE,
    num_subcores=_NSUB,
)


_U = 6
_CPS = 2496 // _NSUB
_NBLK = _CPS // _U


@functools.partial(
    pl.kernel,
    out_type=jax.ShapeDtypeStruct((2 * _N, _D), jnp.float32),
    mesh=_sc_mesh,
    scratch_types=[
        pltpu.VMEM((_U * _C,), jnp.int32),
        pltpu.VMEM((_U, _C), jnp.int32),
        pltpu.VMEM((3, _C, _D), jnp.float32),
        pltpu.VMEM_SHARED((_N, _D), jnp.float32),
        pltpu.SemaphoreType.DMA,
        pltpu.SemaphoreType.DMA((_U,)),
        pltpu.SemaphoreType.DMA((3,)),
        pltpu.SemaphoreType.DMA((3,)),
    ],
)
def _sc_segment(tab, srca, dsta, out, srcb, dstb, rows, accum,
                ssem, dsem, gsem, csem):
    c = lax.axis_index("c")
    s = lax.axis_index("s")

    def _zrow(i, carry):
        for j in range(_D // 16):
            rows[0, i, pl.ds(j * 16, 16)] = jnp.zeros((16,), jnp.float32)
        return carry

    lax.fori_loop(0, _C, _zrow, 0)
    for k in range(4):
        pltpu.sync_copy(rows.at[0], accum.at[pl.ds(s * _RPS + k * _C, _C)])
    pltpu.sync_copy(rows.at[0], accum.at[pl.ds(s * _RPS + _RPS - _C, _C)])

    @pl.when(s == 0)
    def _zero_rem():
        pltpu.sync_copy(rows.at[0], accum.at[pl.ds(_N - _C, _C)])

    plsc.subcore_barrier()

    def _block(jb, carry):
        off0 = (s * _CPS + jb * _U) * _C
        sd = pltpu.async_copy(
            srca.at[pl.ds(c * _E + off0, _U * _C)], srcb, ssem
        )
        dds = [
            pltpu.async_copy(dsta.at[pl.ds(off0 + u * _C, _C)], dstb.at[u],
                             dsem.at[u])
            for u in range(_U)
        ]
        sd.wait()
        g = [None] * _U
        cds = [None] * _U
        g[0] = pltpu.async_copy(tab.at[srcb.at[pl.ds(0, _C)]], rows.at[0],
                                gsem.at[0])
        g[1] = pltpu.async_copy(tab.at[srcb.at[pl.ds(_C, _C)]], rows.at[1],
                                gsem.at[1])
        g[2] = pltpu.async_copy(tab.at[srcb.at[pl.ds(2 * _C, _C)]],
                                rows.at[2], gsem.at[2])
        for u in range(_U):
            g[u].wait()
            dds[u].wait()
            cds[u] = pltpu.async_copy(rows.at[u % 3], accum.at[dstb.at[u]],
                                      csem.at[u % 3], add=True)
            if 1 <= u and u + 2 < _U:
                cds[u - 1].wait()
                g[u + 2] = pltpu.async_copy(
                    tab.at[srcb.at[pl.ds((u + 2) * _C, _C)]],
                    rows.at[(u + 2) % 3], gsem.at[(u + 2) % 3],
                )
        cds[_U - 3].wait()
        cds[_U - 2].wait()
        cds[_U - 1].wait()
        return carry

    lax.fori_loop(0, _NBLK, _block, 0)

    @pl.when(s < _NCH - _CPS * _NSUB)
    def _tail():
        off = (_CPS * _NSUB + s) * _C
        pltpu.async_copy(
            srca.at[pl.ds(c * _E + off, _C)], srcb.at[pl.ds(0, _C)], ssem
        ).wait()
        pltpu.async_copy(dsta.at[pl.ds(off, _C)], dstb.at[0], dsem.at[0]).wait()
        pltpu.async_copy(
            tab.at[srcb.at[pl.ds(0, _C)]], rows.at[0], gsem.at[0]
        ).wait()
        pltpu.async_copy(
            rows.at[0], accum.at[dstb.at[0]], csem.at[0], add=True
        ).wait()

    plsc.subcore_barrier()

    coff = c * _N
    pltpu.sync_copy(
        accum.at[pl.ds(s * _RPS, _RPS)],
        out.at[pl.ds(coff + s * _RPS, _RPS)],
    )

    @pl.when(s == 0)
    def _write_rem():
        pltpu.sync_copy(
            accum.at[pl.ds(_RPS * _NSUB, _REM)],
            out.at[pl.ds(coff + _RPS * _NSUB, _REM)],
        )


def kernel(x, edge_index, enc_W, enc_b, ln_w, ln_b, t, W1, b1, mlp_ln_w,
           mlp_ln_b, W2, b2, lin_W, lin_b, out_W, out_b):
    src = edge_index[0]
    dst = edge_index[1]
    srcsh = jnp.concatenate([src, src + _N])
    h, z, tab = _fin(
        x, enc_W, enc_b.reshape(1, _D),
        ln_w[0].reshape(1, _D), ln_b[0].reshape(1, _D), t[0].reshape(1, 1),
    )
    for i in range(_L - 1):
        s12 = _sc_segment(tab, srcsh, dst)
        h, z, tab = _fmid(
            h, z, s12,
            W1[i], b1[i].reshape(1, _H),
            mlp_ln_w[i].reshape(1, _H), mlp_ln_b[i].reshape(1, _H),
            W2[i], b2[i].reshape(1, _D),
            ln_w[i + 1].reshape(1, _D), ln_b[i + 1].reshape(1, _D),
            t[i + 1].reshape(1, 1),
        )
    s12 = _sc_segment(tab, srcsh, dst)
    return _fout(
        h, z, s12,
        W1[_L - 1], b1[_L - 1].reshape(1, _H),
        mlp_ln_w[_L - 1].reshape(1, _H), mlp_ln_b[_L - 1].reshape(1, _H),
        W2[_L - 1], b2[_L - 1].reshape(1, _D),
        lin_W, lin_b.reshape(1, _D), out_W, out_b.reshape(1, _D),
    )


# speedup vs baseline: 1.0804x; 1.0804x over previous
import functools

import jax
import jax.numpy as jnp
from jax import lax
from jax.experimental import pallas as pl
from jax.experimental.pallas import tpu as pltpu
from jax.experimental.pallas import tpu_sc as plsc

_N = 10000
_E = 320000
_D = 128
_H = 256
_L = 3

_C = 128
_NCH = _E // _C
_NSUB = 16
_NCORE = 2
_RPS = 624
_REM = _N - _RPS * _NSUB


def _graph_ln(h, w, b, eps=1e-5):
    mu = jnp.mean(h)
    var = jnp.mean((h - mu) ** 2)
    return (h - mu) / (jnp.sqrt(var) + eps) * w + b


def _emit_tables(h, lnw, lnb, t, z_ref, tab_ref):
    z = jnp.maximum(_graph_ln(h, lnw, lnb), 0.0)
    z_ref[...] = z
    msg = z + 1e-7
    a = jnp.exp(msg * t)
    tab_ref[0:_N, :] = a
    tab_ref[_N : 2 * _N, :] = msg * a


def _conv_mlp(h_ref, z_ref, s_ref, w1_ref, b1_ref, lnw_ref, lnb_ref,
              w2_ref, b2_ref):
    s1 = s_ref[0:_N, :]
    s2 = s_ref[_N : 2 * _N, :]
    out = s2 / (s1 + 1e-16) + z_ref[...]
    h1 = (
        jnp.dot(out, w1_ref[...], preferred_element_type=jnp.float32)
        + b1_ref[...]
    )
    g = jnp.maximum(_graph_ln(h1, lnw_ref[...], lnb_ref[...]), 0.0)
    return (
        h_ref[...]
        + jnp.dot(g, w2_ref[...], preferred_element_type=jnp.float32)
        + b2_ref[...]
    )


def _fin_body(x_ref, w_ref, b_ref, lnw_ref, lnb_ref, t_ref,
              h_ref, z_ref, tab_ref):
    h = (
        jnp.dot(x_ref[...], w_ref[...], preferred_element_type=jnp.float32)
        + b_ref[...]
    )
    h_ref[...] = h
    _emit_tables(h, lnw_ref[...], lnb_ref[...], t_ref[0, 0], z_ref, tab_ref)


def _fmid_body(h_ref, z_ref, s_ref, w1_ref, b1_ref, lnw_ref, lnb_ref,
               w2_ref, b2_ref, lnwn_ref, lnbn_ref, tn_ref,
               ho_ref, zo_ref, tab_ref):
    h = _conv_mlp(h_ref, z_ref, s_ref, w1_ref, b1_ref, lnw_ref, lnb_ref,
                  w2_ref, b2_ref)
    ho_ref[...] = h
    _emit_tables(h, lnwn_ref[...], lnbn_ref[...], tn_ref[0, 0], zo_ref,
                 tab_ref)


def _fout_body(h_ref, z_ref, s_ref, w1_ref, b1_ref, lnw_ref, lnb_ref,
               w2_ref, b2_ref, lw_ref, lb_ref, ow_ref, ob_ref, o_ref):
    h = _conv_mlp(h_ref, z_ref, s_ref, w1_ref, b1_ref, lnw_ref, lnb_ref,
                  w2_ref, b2_ref)
    g = jnp.maximum(
        jnp.dot(h, lw_ref[...], preferred_element_type=jnp.float32)
        + lb_ref[...],
        0.0,
    )
    o_ref[...] = (
        jnp.dot(g, ow_ref[...], preferred_element_type=jnp.float32)
        + ob_ref[...]
    )


_hzt_shapes = (
    jax.ShapeDtypeStruct((_N, _D), jnp.float32),
    jax.ShapeDtypeStruct((_N, _D), jnp.float32),
    jax.ShapeDtypeStruct((2 * _N, _D), jnp.float32),
)

_fin = pl.pallas_call(_fin_body, out_shape=_hzt_shapes)
_fmid = pl.pallas_call(_fmid_body, out_shape=_hzt_shapes)
_fout = pl.pallas_call(
    _fout_body, out_shape=jax.ShapeDtypeStruct((_N, _D), jnp.float32)
)


_sc_mesh = plsc.VectorSubcoreMesh(
    core_axis_name="c", subcore_axis_name="s", num_cores=_NCORE,
    num_subcores=_NSUB,
)


_U = 6
_CPS = 2496 // _NSUB
_NBLK = _CPS // _U


@functools.partial(
    pl.kernel,
    out_type=jax.ShapeDtypeStruct((2 * _N, _D), jnp.float32),
    mesh=_sc_mesh,
    scratch_types=[
        pltpu.VMEM((_U * _C,), jnp.int32),
        pltpu.VMEM((_U, _C), jnp.int32),
        pltpu.VMEM((3, _C, _D), jnp.float32),
        pltpu.VMEM_SHARED((_N, _D), jnp.float32),
        pltpu.SemaphoreType.DMA,
        pltpu.SemaphoreType.DMA((_U,)),
        pltpu.SemaphoreType.DMA((3,)),
        pltpu.SemaphoreType.DMA((3,)),
    ],
)
def _sc_segment(tab, srca, dsta, out, srcb, dstb, rows, accum,
                ssem, dsem, gsem, csem):
    c = lax.axis_index("c")
    s = lax.axis_index("s")

    def _zrow(i, carry):
        for j in range(_D // 16):
            rows[0, i, pl.ds(j * 16, 16)] = jnp.zeros((16,), jnp.float32)
        return carry

    lax.fori_loop(0, _C, _zrow, 0)
    for k in range(4):
        pltpu.sync_copy(rows.at[0], accum.at[pl.ds(s * _RPS + k * _C, _C)])
    pltpu.sync_copy(rows.at[0], accum.at[pl.ds(s * _RPS + _RPS - _C, _C)])

    @pl.when(s == 0)
    def _zero_rem():
        pltpu.sync_copy(rows.at[0], accum.at[pl.ds(_N - _C, _C)])

    plsc.subcore_barrier()

    def _block(jb, carry):
        off0 = (s * _CPS + jb * _U) * _C
        sd = pltpu.async_copy(
            srca.at[pl.ds(c * _E + off0, _U * _C)], srcb, ssem
        )
        dds = [
            pltpu.async_copy(dsta.at[pl.ds(off0 + u * _C, _C)], dstb.at[u],
                             dsem.at[u])
            for u in range(_U)
        ]
        sd.wait()
        g = [None] * _U
        cds = [None] * _U
        g[0] = pltpu.async_copy(tab.at[srcb.at[pl.ds(0, _C)]], rows.at[0],
                                gsem.at[0])
        g[1] = pltpu.async_copy(tab.at[srcb.at[pl.ds(_C, _C)]], rows.at[1],
                                gsem.at[1])
        for u in range(_U):
            g[u].wait()
            dds[u].wait()
            cds[u] = pltpu.async_copy(rows.at[u % 2], accum.at[dstb.at[u]],
                                      csem.at[u % 2], add=True)
            if u + 2 < _U:
                cds[u].wait()
                g[u + 2] = pltpu.async_copy(
                    tab.at[srcb.at[pl.ds((u + 2) * _C, _C)]],
                    rows.at[u % 2], gsem.at[u % 2],
                )
        cds[_U - 2].wait()
        cds[_U - 1].wait()
        return carry

    lax.fori_loop(0, _NBLK, _block, 0)

    @pl.when(s < _NCH - _CPS * _NSUB)
    def _tail():
        off = (_CPS * _NSUB + s) * _C
        pltpu.async_copy(
            srca.at[pl.ds(c * _E + off, _C)], srcb.at[pl.ds(0, _C)], ssem
        ).wait()
        pltpu.async_copy(dsta.at[pl.ds(off, _C)], dstb.at[0], dsem.at[0]).wait()
        pltpu.async_copy(
            tab.at[srcb.at[pl.ds(0, _C)]], rows.at[0], gsem.at[0]
        ).wait()
        pltpu.async_copy(
            rows.at[0], accum.at[dstb.at[0]], csem.at[0], add=True
        ).wait()

    plsc.subcore_barrier()

    coff = c * _N
    pltpu.sync_copy(
        accum.at[pl.ds(s * _RPS, _RPS)],
        out.at[pl.ds(coff + s * _RPS, _RPS)],
    )

    @pl.when(s == 0)
    def _write_rem():
        pltpu.sync_copy(
            accum.at[pl.ds(_RPS * _NSUB, _REM)],
            out.at[pl.ds(coff + _RPS * _NSUB, _REM)],
        )


def kernel(x, edge_index, enc_W, enc_b, ln_w, ln_b, t, W1, b1, mlp_ln_w,
           mlp_ln_b, W2, b2, lin_W, lin_b, out_W, out_b):
    src = edge_index[0]
    dst = edge_index[1]
    srcsh = jnp.concatenate([src, src + _N])
    h, z, tab = _fin(
        x, enc_W, enc_b.reshape(1, _D),
        ln_w[0].reshape(1, _D), ln_b[0].reshape(1, _D), t[0].reshape(1, 1),
    )
    for i in range(_L - 1):
        s12 = _sc_segment(tab, srcsh, dst)
        h, z, tab = _fmid(
            h, z, s12,
            W1[i], b1[i].reshape(1, _H),
            mlp_ln_w[i].reshape(1, _H), mlp_ln_b[i].reshape(1, _H),
            W2[i], b2[i].reshape(1, _D),
            ln_w[i + 1].reshape(1, _D), ln_b[i + 1].reshape(1, _D),
            t[i + 1].reshape(1, 1),
        )
    s12 = _sc_segment(tab, srcsh, dst)
    return _fout(
        h, z, s12,
        W1[_L - 1], b1[_L - 1].reshape(1, _H),
        mlp_ln_w[_L - 1].reshape(1, _H), mlp_ln_b[_L - 1].reshape(1, _H),
        W2[_L - 1], b2[_L - 1].reshape(1, _D),
        lin_W, lin_b.reshape(1, _D), out_W, out_b.reshape(1, _D),
    )


# speedup vs baseline: 1.2281x; 1.1367x over previous
import functools

import jax
import jax.numpy as jnp
from jax import lax
from jax.experimental import pallas as pl
from jax.experimental.pallas import tpu as pltpu
from jax.experimental.pallas import tpu_sc as plsc

_N = 10000
_E = 320000
_D = 128
_H = 256
_L = 3

_C = 128
_NCH = _E // _C
_NSUB = 16
_NCORE = 2
_RPS = 624
_REM = _N - _RPS * _NSUB


def _graph_ln(h, w, b, eps=1e-5):
    mu = jnp.mean(h)
    var = jnp.mean((h - mu) ** 2)
    return (h - mu) / (jnp.sqrt(var) + eps) * w + b


def _emit_tables(h, lnw, lnb, t, z_ref, tab_ref):
    z = jnp.maximum(_graph_ln(h, lnw, lnb), 0.0)
    z_ref[...] = z
    msg = z + 1e-7
    a = jnp.exp(msg * t)
    tab_ref[0:_N, :] = a
    tab_ref[_N : 2 * _N, :] = msg * a


def _conv_mlp(h_ref, z_ref, s_ref, w1_ref, b1_ref, lnw_ref, lnb_ref,
              w2_ref, b2_ref):
    s1 = s_ref[0:_N, :]
    s2 = s_ref[_N : 2 * _N, :]
    out = s2 / (s1 + 1e-16) + z_ref[...]
    h1 = (
        jnp.dot(out, w1_ref[...], preferred_element_type=jnp.float32)
        + b1_ref[...]
    )
    g = jnp.maximum(_graph_ln(h1, lnw_ref[...], lnb_ref[...]), 0.0)
    return (
        h_ref[...]
        + jnp.dot(g, w2_ref[...], preferred_element_type=jnp.float32)
        + b2_ref[...]
    )


def _fin_body(x_ref, w_ref, b_ref, lnw_ref, lnb_ref, t_ref,
              h_ref, z_ref, tab_ref):
    h = (
        jnp.dot(x_ref[...], w_ref[...], preferred_element_type=jnp.float32)
        + b_ref[...]
    )
    h_ref[...] = h
    _emit_tables(h, lnw_ref[...], lnb_ref[...], t_ref[0, 0], z_ref, tab_ref)


def _fmid_body(h_ref, z_ref, s_ref, w1_ref, b1_ref, lnw_ref, lnb_ref,
               w2_ref, b2_ref, lnwn_ref, lnbn_ref, tn_ref,
               ho_ref, zo_ref, tab_ref):
    h = _conv_mlp(h_ref, z_ref, s_ref, w1_ref, b1_ref, lnw_ref, lnb_ref,
                  w2_ref, b2_ref)
    ho_ref[...] = h
    _emit_tables(h, lnwn_ref[...], lnbn_ref[...], tn_ref[0, 0], zo_ref,
                 tab_ref)


def _fout_body(h_ref, z_ref, s_ref, w1_ref, b1_ref, lnw_ref, lnb_ref,
               w2_ref, b2_ref, lw_ref, lb_ref, ow_ref, ob_ref, o_ref):
    h = _conv_mlp(h_ref, z_ref, s_ref, w1_ref, b1_ref, lnw_ref, lnb_ref,
                  w2_ref, b2_ref)
    g = jnp.maximum(
        jnp.dot(h, lw_ref[...], preferred_element_type=jnp.float32)
        + lb_ref[...],
        0.0,
    )
    o_ref[...] = (
        jnp.dot(g, ow_ref[...], preferred_element_type=jnp.float32)
        + ob_ref[...]
    )


_hzt_shapes = (
    jax.ShapeDtypeStruct((_N, _D), jnp.float32),
    jax.ShapeDtypeStruct((_N, _D), jnp.float32),
    jax.ShapeDtypeStruct((2 * _N, _D), jnp.float32),
)

_fin = pl.pallas_call(_fin_body, out_shape=_hzt_shapes)
_fmid = pl.pallas_call(_fmid_body, out_shape=_hzt_shapes)
_fout = pl.pallas_call(
    _fout_body, out_shape=jax.ShapeDtypeStruct((_N, _D), jnp.float32)
)


_sc_mesh = plsc.VectorSubcoreMesh(
    core_axis_name="c", subcore_axis_name="s", num_cores=_NCORE,
    num_subcores=_NSUB,
)


_U = 6
_CPS = 2496 // _NSUB
_NBLK = _CPS // _U


@functools.partial(
    pl.kernel,
    out_type=jax.ShapeDtypeStruct((2 * _N, _D), jnp.float32),
    mesh=_sc_mesh,
    scratch_types=[
        pltpu.VMEM((2, _U * _C), jnp.int32),
        pltpu.VMEM((2, _U, _C), jnp.int32),
        pltpu.VMEM((2, _C, _D), jnp.float32),
        pltpu.VMEM_SHARED((_N, _D), jnp.float32),
        pltpu.SemaphoreType.DMA((2,)),
        pltpu.SemaphoreType.DMA((2 * _U,)),
        pltpu.SemaphoreType.DMA((2,)),
        pltpu.SemaphoreType.DMA((2,)),
    ],
)
def _sc_segment(tab, srca, dsta, out, srcb, dstb, rows, accum,
                ssem, dsem, gsem, csem):
    c = lax.axis_index("c")
    s = lax.axis_index("s")

    def _zrow(i, carry):
        for j in range(_D // 16):
            rows[0, i, pl.ds(j * 16, 16)] = jnp.zeros((16,), jnp.float32)
        return carry

    lax.fori_loop(0, _C, _zrow, 0)
    for k in range(4):
        pltpu.sync_copy(rows.at[0], accum.at[pl.ds(s * _RPS + k * _C, _C)])
    pltpu.sync_copy(rows.at[0], accum.at[pl.ds(s * _RPS + _RPS - _C, _C)])

    @pl.when(s == 0)
    def _zero_rem():
        pltpu.sync_copy(rows.at[0], accum.at[pl.ds(_N - _C, _C)])

    plsc.subcore_barrier()

    def _idx_issue(p, jb):
        off0 = (s * _CPS + jb * _U) * _C
        pltpu.async_copy(
            srca.at[pl.ds(c * _E + off0, _U * _C)], srcb.at[p], ssem.at[p]
        )
        for u in range(_U):
            pltpu.async_copy(dsta.at[pl.ds(off0 + u * _C, _C)],
                             dstb.at[p, u], dsem.at[p * _U + u])

    def _gather(p, u, k):
        return pltpu.async_copy(
            tab.at[srcb.at[p, pl.ds(u * _C, _C)]], rows.at[k], gsem.at[k]
        )

    def _wait_recon(src_ref, dst_ref, sem):
        pltpu.make_async_copy(src_ref, dst_ref, sem).wait()

    def _process(p, jb, issue_next):
        q = 1 - p
        if issue_next:
            _idx_issue(q, jb + 1)
        g = [None] * _U
        cds = [None] * _U
        for u in range(_U):
            if u < 2:
                _wait_recon(tab.at[pl.ds(0, _C)], rows.at[u], gsem.at[u])
            else:
                g[u].wait()
            _wait_recon(dsta.at[pl.ds(0, _C)], dstb.at[p, u],
                        dsem.at[p * _U + u])
            cds[u] = pltpu.async_copy(rows.at[u % 2], accum.at[dstb.at[p, u]],
                                      csem.at[u % 2], add=True)
            if u + 2 < _U:
                cds[u].wait()
                g[u + 2] = _gather(p, u + 2, u % 2)
        cds[_U - 2].wait()
        if issue_next:
            _wait_recon(srca.at[pl.ds(0, _U * _C)], srcb.at[q], ssem.at[q])
            _gather(q, 0, 0)
        cds[_U - 1].wait()
        if issue_next:
            _gather(q, 1, 1)

    _idx_issue(0, 0)
    _wait_recon(srca.at[pl.ds(0, _U * _C)], srcb.at[0], ssem.at[0])
    _gather(0, 0, 0)
    _gather(0, 1, 1)

    def _pair(jb2, carry):
        _process(0, 2 * jb2, True)
        _process(1, 2 * jb2 + 1, True)
        return carry

    lax.fori_loop(0, _NBLK // 2 - 1, _pair, 0)
    _process(0, _NBLK - 2, True)
    _process(1, _NBLK - 1, False)

    @pl.when(s < _NCH - _CPS * _NSUB)
    def _tail():
        off = (_CPS * _NSUB + s) * _C
        pltpu.async_copy(
            srca.at[pl.ds(c * _E + off, _C)], srcb.at[0, pl.ds(0, _C)],
            ssem.at[0],
        ).wait()
        pltpu.async_copy(dsta.at[pl.ds(off, _C)], dstb.at[0, 0],
                         dsem.at[0]).wait()
        pltpu.async_copy(
            tab.at[srcb.at[0, pl.ds(0, _C)]], rows.at[0], gsem.at[0]
        ).wait()
        pltpu.async_copy(
            rows.at[0], accum.at[dstb.at[0, 0]], csem.at[0], add=True
        ).wait()

    plsc.subcore_barrier()

    coff = c * _N
    pltpu.sync_copy(
        accum.at[pl.ds(s * _RPS, _RPS)],
        out.at[pl.ds(coff + s * _RPS, _RPS)],
    )

    @pl.when(s == 0)
    def _write_rem():
        pltpu.sync_copy(
            accum.at[pl.ds(_RPS * _NSUB, _REM)],
            out.at[pl.ds(coff + _RPS * _NSUB, _REM)],
        )


def kernel(x, edge_index, enc_W, enc_b, ln_w, ln_b, t, W1, b1, mlp_ln_w,
           mlp_ln_b, W2, b2, lin_W, lin_b, out_W, out_b):
    src = edge_index[0]
    dst = edge_index[1]
    srcsh = jnp.concatenate([src, src + _N])
    h, z, tab = _fin(
        x, enc_W, enc_b.reshape(1, _D),
        ln_w[0].reshape(1, _D), ln_b[0].reshape(1, _D), t[0].reshape(1, 1),
    )
    for i in range(_L - 1):
        s12 = _sc_segment(tab, srcsh, dst)
        h, z, tab = _fmid(
            h, z, s12,
            W1[i], b1[i].reshape(1, _H),
            mlp_ln_w[i].reshape(1, _H), mlp_ln_b[i].reshape(1, _H),
            W2[i], b2[i].reshape(1, _D),
            ln_w[i + 1].reshape(1, _D), ln_b[i + 1].reshape(1, _D),
            t[i + 1].reshape(1, 1),
        )
    s12 = _sc_segment(tab, srcsh, dst)
    return _fout(
        h, z, s12,
        W1[_L - 1], b1[_L - 1].reshape(1, _H),
        mlp_ln_w[_L - 1].reshape(1, _H), mlp_ln_b[_L - 1].reshape(1, _H),
        W2[_L - 1], b2[_L - 1].reshape(1, _D),
        lin_W, lin_b.reshape(1, _D), out_W, out_b.reshape(1, _D),
    )


# speedup vs baseline: 1.2338x; 1.0046x over previous
import functools

import jax
import jax.numpy as jnp
from jax import lax
from jax.experimental import pallas as pl
from jax.experimental.pallas import tpu as pltpu
from jax.experimental.pallas import tpu_sc as plsc

_N = 10000
_E = 320000
_D = 128
_H = 256
_L = 3

_C = 128
_NCH = _E // _C
_NSUB = 16
_NCORE = 2
_RPS = 624
_REM = _N - _RPS * _NSUB


def _graph_ln(h, w, b, eps=1e-5):
    mu = jnp.mean(h)
    var = jnp.mean((h - mu) ** 2)
    return (h - mu) / (jnp.sqrt(var) + eps) * w + b


def _emit_tables(h, lnw, lnb, t, z_ref, tab_ref):
    z = jnp.maximum(_graph_ln(h, lnw, lnb), 0.0)
    z_ref[...] = z
    msg = z + 1e-7
    a = jnp.exp(msg * t)
    tab_ref[0:_N, :] = a
    tab_ref[_N : 2 * _N, :] = msg * a


def _conv_mlp(h_ref, z_ref, s_ref, w1_ref, b1_ref, lnw_ref, lnb_ref,
              w2_ref, b2_ref):
    s1 = s_ref[0:_N, :]
    s2 = s_ref[_N : 2 * _N, :]
    out = s2 / (s1 + 1e-16) + z_ref[...]
    h1 = (
        jnp.dot(out, w1_ref[...], preferred_element_type=jnp.float32)
        + b1_ref[...]
    )
    g = jnp.maximum(_graph_ln(h1, lnw_ref[...], lnb_ref[...]), 0.0)
    return (
        h_ref[...]
        + jnp.dot(g, w2_ref[...], preferred_element_type=jnp.float32)
        + b2_ref[...]
    )


def _fin_body(x_ref, w_ref, b_ref, lnw_ref, lnb_ref, t_ref,
              h_ref, z_ref, tab_ref):
    h = (
        jnp.dot(x_ref[...], w_ref[...], preferred_element_type=jnp.float32)
        + b_ref[...]
    )
    h_ref[...] = h
    _emit_tables(h, lnw_ref[...], lnb_ref[...], t_ref[0, 0], z_ref, tab_ref)


def _fmid_body(h_ref, z_ref, s_ref, w1_ref, b1_ref, lnw_ref, lnb_ref,
               w2_ref, b2_ref, lnwn_ref, lnbn_ref, tn_ref,
               ho_ref, zo_ref, tab_ref):
    h = _conv_mlp(h_ref, z_ref, s_ref, w1_ref, b1_ref, lnw_ref, lnb_ref,
                  w2_ref, b2_ref)
    ho_ref[...] = h
    _emit_tables(h, lnwn_ref[...], lnbn_ref[...], tn_ref[0, 0], zo_ref,
                 tab_ref)


def _fout_body(h_ref, z_ref, s_ref, w1_ref, b1_ref, lnw_ref, lnb_ref,
               w2_ref, b2_ref, lw_ref, lb_ref, ow_ref, ob_ref, o_ref):
    h = _conv_mlp(h_ref, z_ref, s_ref, w1_ref, b1_ref, lnw_ref, lnb_ref,
                  w2_ref, b2_ref)
    g = jnp.maximum(
        jnp.dot(h, lw_ref[...], preferred_element_type=jnp.float32)
        + lb_ref[...],
        0.0,
    )
    o_ref[...] = (
        jnp.dot(g, ow_ref[...], preferred_element_type=jnp.float32)
        + ob_ref[...]
    )


_hzt_shapes = (
    jax.ShapeDtypeStruct((_N, _D), jnp.float32),
    jax.ShapeDtypeStruct((_N, _D), jnp.float32),
    jax.ShapeDtypeStruct((2 * _N, _D), jnp.float32),
)

_fin = pl.pallas_call(_fin_body, out_shape=_hzt_shapes)
_fmid = pl.pallas_call(_fmid_body, out_shape=_hzt_shapes)
_fout = pl.pallas_call(
    _fout_body, out_shape=jax.ShapeDtypeStruct((_N, _D), jnp.float32)
)


_sc_mesh = plsc.VectorSubcoreMesh(
    core_axis_name="c", subcore_axis_name="s", num_cores=_NCORE,
    num_subcores=_NSUB,
)


_U = 6
_CPS = 2496 // _NSUB
_NBLK = _CPS // _U


@functools.partial(
    pl.kernel,
    out_type=jax.ShapeDtypeStruct((2 * _N, _D), jnp.float32),
    mesh=_sc_mesh,
    scratch_types=[
        pltpu.VMEM((2, _U * _C), jnp.int32),
        pltpu.VMEM((2, _U, _C), jnp.int32),
        pltpu.VMEM((2, _C, _D), jnp.float32),
        pltpu.VMEM_SHARED((_N, _D), jnp.float32),
        pltpu.SemaphoreType.DMA((2,)),
        pltpu.SemaphoreType.DMA((2 * _U,)),
        pltpu.SemaphoreType.DMA((2,)),
        pltpu.SemaphoreType.DMA((2,)),
    ],
)
def _sc_segment(tab, srca, dsta, out, srcb, dstb, rows, accum,
                ssem, dsem, gsem, csem):
    c = lax.axis_index("c")
    s = lax.axis_index("s")

    def _zrow(i, carry):
        for j in range(_D // 16):
            rows[0, i, pl.ds(j * 16, 16)] = jnp.zeros((16,), jnp.float32)
        return carry

    def _idx_issue(p, jb):
        off0 = (s * _CPS + jb * _U) * _C
        pltpu.async_copy(
            srca.at[pl.ds(c * _E + off0, _U * _C)], srcb.at[p], ssem.at[p]
        )
        for u in range(_U):
            pltpu.async_copy(dsta.at[pl.ds(off0 + u * _C, _C)],
                             dstb.at[p, u], dsem.at[p * _U + u])

    def _gather(p, u, k):
        return pltpu.async_copy(
            tab.at[srcb.at[p, pl.ds(u * _C, _C)]], rows.at[k], gsem.at[k]
        )

    def _wait_recon(src_ref, dst_ref, sem):
        pltpu.make_async_copy(src_ref, dst_ref, sem).wait()

    def _process(p, jb, issue_next):
        q = 1 - p
        if issue_next:
            _idx_issue(q, jb + 1)
        g = [None] * _U
        cds = [None] * _U
        for u in range(_U):
            if u < 2:
                _wait_recon(tab.at[pl.ds(0, _C)], rows.at[u], gsem.at[u])
            else:
                g[u].wait()
            _wait_recon(dsta.at[pl.ds(0, _C)], dstb.at[p, u],
                        dsem.at[p * _U + u])
            cds[u] = pltpu.async_copy(rows.at[u % 2], accum.at[dstb.at[p, u]],
                                      csem.at[u % 2], add=True)
            if u + 2 < _U:
                cds[u].wait()
                g[u + 2] = _gather(p, u + 2, u % 2)
        cds[_U - 2].wait()
        if issue_next:
            _wait_recon(srca.at[pl.ds(0, _U * _C)], srcb.at[q], ssem.at[q])
            _gather(q, 0, 0)
        cds[_U - 1].wait()
        if issue_next:
            _gather(q, 1, 1)

    _idx_issue(0, 0)
    lax.fori_loop(0, _C, _zrow, 0)
    for k in range(4):
        pltpu.sync_copy(rows.at[0], accum.at[pl.ds(s * _RPS + k * _C, _C)])
    pltpu.sync_copy(rows.at[0], accum.at[pl.ds(s * _RPS + _RPS - _C, _C)])

    @pl.when(s == 0)
    def _zero_rem():
        pltpu.sync_copy(rows.at[0], accum.at[pl.ds(_N - _C, _C)])

    _wait_recon(srca.at[pl.ds(0, _U * _C)], srcb.at[0], ssem.at[0])
    _gather(0, 0, 0)
    _gather(0, 1, 1)
    plsc.subcore_barrier()

    def _pair(jb2, carry):
        _process(0, 2 * jb2, True)
        _process(1, 2 * jb2 + 1, True)
        return carry

    lax.fori_loop(0, _NBLK // 2 - 1, _pair, 0)
    _process(0, _NBLK - 2, True)
    _process(1, _NBLK - 1, False)

    @pl.when(s < _NCH - _CPS * _NSUB)
    def _tail():
        off = (_CPS * _NSUB + s) * _C
        pltpu.async_copy(
            srca.at[pl.ds(c * _E + off, _C)], srcb.at[0, pl.ds(0, _C)],
            ssem.at[0],
        ).wait()
        pltpu.async_copy(dsta.at[pl.ds(off, _C)], dstb.at[0, 0],
                         dsem.at[0]).wait()
        pltpu.async_copy(
            tab.at[srcb.at[0, pl.ds(0, _C)]], rows.at[0], gsem.at[0]
        ).wait()
        pltpu.async_copy(
            rows.at[0], accum.at[dstb.at[0, 0]], csem.at[0], add=True
        ).wait()

    plsc.subcore_barrier()

    coff = c * _N
    pltpu.sync_copy(
        accum.at[pl.ds(s * _RPS, _RPS)],
        out.at[pl.ds(coff + s * _RPS, _RPS)],
    )

    @pl.when(s == 0)
    def _write_rem():
        pltpu.sync_copy(
            accum.at[pl.ds(_RPS * _NSUB, _REM)],
            out.at[pl.ds(coff + _RPS * _NSUB, _REM)],
        )


def kernel(x, edge_index, enc_W, enc_b, ln_w, ln_b, t, W1, b1, mlp_ln_w,
           mlp_ln_b, W2, b2, lin_W, lin_b, out_W, out_b):
    src = edge_index[0]
    dst = edge_index[1]
    srcsh = jnp.concatenate([src, src + _N])
    h, z, tab = _fin(
        x, enc_W, enc_b.reshape(1, _D),
        ln_w[0].reshape(1, _D), ln_b[0].reshape(1, _D), t[0].reshape(1, 1),
    )
    for i in range(_L - 1):
        s12 = _sc_segment(tab, srcsh, dst)
        h, z, tab = _fmid(
            h, z, s12,
            W1[i], b1[i].reshape(1, _H),
            mlp_ln_w[i].reshape(1, _H), mlp_ln_b[i].reshape(1, _H),
            W2[i], b2[i].reshape(1, _D),
            ln_w[i + 1].reshape(1, _D), ln_b[i + 1].reshape(1, _D),
            t[i + 1].reshape(1, 1),
        )
    s12 = _sc_segment(tab, srcsh, dst)
    return _fout(
        h, z, s12,
        W1[_L - 1], b1[_L - 1].reshape(1, _H),
        mlp_ln_w[_L - 1].reshape(1, _H), mlp_ln_b[_L - 1].reshape(1, _H),
        W2[_L - 1], b2[_L - 1].reshape(1, _D),
        lin_W, lin_b.reshape(1, _D), out_W, out_b.reshape(1, _D),
    )
